# baseline (speedup 1.0000x reference)
import functools

import jax
import jax.numpy as jnp
from jax import lax
from jax.experimental import pallas as pl
from jax.experimental.pallas import tpu as pltpu
from jax.experimental.pallas import tpu_sc as plsc

LANES = 16
NBUF = 3


@functools.lru_cache(maxsize=None)
def _make_emb_kernel(batch, seq, vocab, d_model):
    info = plsc.get_sparse_core_info()
    nc, ns = info.num_cores, info.num_subcores
    nw = nc * ns
    assert seq % nw == 0
    s_per_w = seq // nw
    ch = 32
    n_halves = s_per_w // ch
    n_chunks = n_halves * batch
    assert d_model % LANES == 0
    cols = d_model // LANES

    mesh = plsc.VectorSubcoreMesh(core_axis_name="c", subcore_axis_name="s")

    @functools.partial(
        pl.kernel,
        mesh=mesh,
        out_type=jax.ShapeDtypeStruct((batch * seq, d_model), jnp.float32),
        scratch_types=(
            [
                pltpu.VMEM((batch, s_per_w), jnp.int32),
                pltpu.VMEM((NBUF, ch, d_model), jnp.float32),
                pltpu.VMEM((s_per_w, d_model), jnp.float32),
            ]
            + [pltpu.SemaphoreType.DMA for _ in range(2 * NBUF + 1)]
        ),
    )
    def emb(idx_hbm, tok_hbm, pos_hbm, out_hbm, idx_v, tok_v, pos_v, *sems):
        gsem = sems[0:NBUF]
        wsem = sems[NBUF:2 * NBUF]
        psem = sems[2 * NBUF]

        wid = lax.axis_index("s") * nc + lax.axis_index("c")
        s0 = wid * s_per_w

        def gather_copy(g, make=False):
            b, h, k = g % NBUF, g // batch, g % batch
            src = tok_hbm.at[idx_v.at[k, pl.ds(h * ch, ch)]]
            f = pltpu.make_async_copy if make else pltpu.async_copy
            return f(src, tok_v.at[b], gsem[b if isinstance(g, int) else 0])

        def out_copy(g, make=False):
            b, h, k = g % NBUF, g // batch, g % batch
            row0 = k * seq + s0 + h * ch
            f = pltpu.make_async_copy if make else pltpu.async_copy
            return f(tok_v.at[b], out_hbm.at[pl.ds(row0, ch)],
                     wsem[b if isinstance(g, int) else 0])

        idx_cps = [
            pltpu.async_copy(
                idx_hbm.at[pl.ds(k * seq + s0, s_per_w)], idx_v.at[k], psem
            )
            for k in range(batch)
        ]
        pos_cp = pltpu.async_copy(pos_hbm.at[pl.ds(s0, s_per_w)], pos_v, psem)
        for c in idx_cps:
            c.wait()
        gather_copy(0)
        gather_copy(1)
        pos_cp.wait()

        def per_buf(b, fn):
            for bb in range(NBUF):
                pl.when(b == bb)(lambda bb=bb: fn(bb))

        def chunk_body(g, carry):
            b, h, k = g % NBUF, g // batch, g % batch

            def wait_gather(bb):
                src = tok_hbm.at[idx_v.at[k, pl.ds(h * ch, ch)]]
                pltpu.make_async_copy(src, tok_v.at[bb], gsem[bb]).wait()

            per_buf(b, wait_gather)

            def row_body(r, carry2):
                pr = h * ch + r
                for c in range(cols):
                    s = c * LANES
                    plsc.addupdate(
                        tok_v.at[b, r, pl.ds(s, LANES)],
                        pos_v[pr, pl.ds(s, LANES)],
                    )
                return carry2

            lax.fori_loop(0, ch, row_body, 0)

            def issue_out(bb):
                row0 = k * seq + s0 + h * ch
                pltpu.async_copy(
                    tok_v.at[bb], out_hbm.at[pl.ds(row0, ch)], wsem[bb]
                )

            per_buf(b, issue_out)

            @pl.when(g + 2 < n_chunks)
            def _prefetch():
                g2 = g + 2
                b2, h2, k2 = g2 % NBUF, g2 // batch, g2 % batch

                def drain_and_gather(bb):
                    @pl.when(g >= 1)
                    def _():
                        gp = g - 1
                        hp, kp = gp // batch, gp % batch
                        row0 = kp * seq + s0 + hp * ch
                        pltpu.make_async_copy(
                            tok_v.at[bb], out_hbm.at[pl.ds(row0, ch)],
                            wsem[bb],
                        ).wait()
                    src = tok_hbm.at[idx_v.at[k2, pl.ds(h2 * ch, ch)]]
                    pltpu.async_copy(src, tok_v.at[bb], gsem[bb])

                per_buf(b2, drain_and_gather)

            return carry

        lax.fori_loop(0, n_chunks, chunk_body, 0)
        for g in range(n_chunks - NBUF, n_chunks):
            out_copy(g, make=True).wait()

    return emb


def kernel(x, token_table, pos_table):
    b, s = x.shape
    v, d = token_table.shape
    idx = x.reshape(b * s).astype(jnp.int32)
    emb = _make_emb_kernel(b, s, v, d)
    out = emb(idx, token_table, pos_table)
    return out.reshape(b, s, d)


# speedup vs baseline: 1.1662x; 1.1662x over previous
import functools

import jax
import jax.numpy as jnp
from jax import lax
from jax.experimental import pallas as pl
from jax.experimental.pallas import tpu as pltpu
from jax.experimental.pallas import tpu_sc as plsc

LANES = 16
NBUF = 3


@functools.lru_cache(maxsize=None)
def _make_emb_kernel(batch, seq, vocab, d_model):
    info = plsc.get_sparse_core_info()
    nc, ns = info.num_cores, info.num_subcores
    nw = nc * ns
    assert seq % nw == 0
    s_per_w = seq // nw
    ch = 32
    n_halves = s_per_w // ch
    n_chunks = n_halves * batch
    assert d_model % LANES == 0
    cols = d_model // LANES

    mesh = plsc.VectorSubcoreMesh(core_axis_name="c", subcore_axis_name="s")

    @functools.partial(
        pl.kernel,
        mesh=mesh,
        out_type=jax.ShapeDtypeStruct((batch, seq, d_model), jnp.float32),
        scratch_types=(
            [pltpu.VMEM((batch, s_per_w), jnp.int32)]
            + [pltpu.VMEM((ch, d_model), jnp.float32) for _ in range(NBUF)]
            + [pltpu.VMEM((s_per_w, d_model), jnp.float32)]
            + [pltpu.SemaphoreType.DMA for _ in range(2 * NBUF + 1)]
        ),
    )
    def emb(x_hbm, tok_hbm, pos_hbm, out_hbm, idx_v, *refs):
        tok_v = refs[0:NBUF]
        pos_v = refs[NBUF]
        gsem = refs[NBUF + 1:2 * NBUF + 1]
        wsem = refs[2 * NBUF + 1:3 * NBUF + 1]
        psem = refs[3 * NBUF + 1]

        wid = lax.axis_index("s") * nc + lax.axis_index("c")
        s0 = wid * s_per_w

        def hk(g):
            return g // batch, g % batch

        def issue_gather(g):
            b = g % NBUF
            h, k = hk(g)
            return pltpu.async_copy(
                tok_hbm.at[idx_v.at[k, pl.ds(h * ch, ch)]], tok_v[b], gsem[b]
            )

        def issue_out(g):
            b = g % NBUF
            h, k = hk(g)
            return pltpu.async_copy(
                tok_v[b], out_hbm.at[k, pl.ds(s0 + h * ch, ch)], wsem[b]
            )

        idx_cps = [
            pltpu.async_copy(
                x_hbm.at[k, pl.ds(s0, s_per_w)], idx_v.at[k], psem
            )
            for k in range(batch)
        ]
        pos_cp = pltpu.async_copy(pos_hbm.at[pl.ds(s0, s_per_w)], pos_v, psem)
        for c in idx_cps:
            c.wait()

        gather_cp = {g: issue_gather(g) for g in range(2)}
        pos_cp.wait()

        out_cp = {}
        for g in range(n_chunks):
            b = g % NBUF
            h, _ = hk(g)
            gather_cp[g].wait()

            def row_body(r, carry, b=b, h=h):
                for c in range(cols):
                    s = c * LANES
                    plsc.addupdate(
                        tok_v[b].at[r, pl.ds(s, LANES)],
                        pos_v[h * ch + r, pl.ds(s, LANES)],
                    )
                return carry

            lax.fori_loop(0, ch, row_body, 0)
            out_cp[g] = issue_out(g)
            if g + 2 < n_chunks:
                if g - 1 >= 0:
                    out_cp[g - 1].wait()
                gather_cp[g + 2] = issue_gather(g + 2)
        for g in range(n_chunks - NBUF, n_chunks):
            out_cp[g].wait()

    return emb


def kernel(x, token_table, pos_table):
    b, s = x.shape
    v, d = token_table.shape
    emb = _make_emb_kernel(b, s, v, d)
    return emb(x, token_table, pos_table)
